# baseline (speedup 1.0000x reference)
import functools

import jax
import jax.numpy as jnp
from jax.experimental import pallas as pl
from jax.experimental.pallas import tpu as pltpu

CH = 256
FB = 512
RPAD = 4608


def _router_body(x_ref, wr_ref, br_ref, g1_ref, g2_ref, i1_ref, i2_ref, *, n_e):
    x = x_ref[...]
    logits = jnp.dot(x, wr_ref[...], preferred_element_type=jnp.float32)
    logits = logits + br_ref[...]
    m = jnp.max(logits, axis=1, keepdims=True)
    ex = jnp.exp(logits - m)
    rw = ex / jnp.sum(ex, axis=1, keepdims=True)
    iota = jax.lax.broadcasted_iota(jnp.int32, rw.shape, 1)
    m1 = jnp.max(rw, axis=1, keepdims=True)
    i1 = jnp.min(jnp.where(rw == m1, iota, n_e), axis=1, keepdims=True)
    oh1 = iota == i1
    rw2 = jnp.where(oh1, -1e30, rw)
    m2 = jnp.max(rw2, axis=1, keepdims=True)
    i2 = jnp.min(jnp.where(rw2 == m2, iota, n_e), axis=1, keepdims=True)
    v1 = jnp.sum(jnp.where(oh1, rw, 0.0), axis=1, keepdims=True)
    v2 = jnp.sum(jnp.where(iota == i2, rw, 0.0), axis=1, keepdims=True)
    tot = v1 + v2
    g1_ref[...] = v1 / tot
    g2_ref[...] = v2 / tot
    i1_ref[...] = i1
    i2_ref[...] = i2


def _ffn_body(off_ref, nck_ref, x_ref, w1_hbm, b1_ref, w2_hbm, b2_ref,
              gate_ref, out_ref, w1buf, w2buf, sem1, sem2, *, nf, n_e):
    e = pl.program_id(0)
    f = pl.program_id(1)
    off = off_ref[e]
    nc = nck_ref[e]
    s = e * nf + f
    slot = jax.lax.rem(s, 2)
    nslot = jax.lax.rem(s + 1, 2)

    def fetch(step, buf_slot):
        en = step // nf
        fn_ = jax.lax.rem(step, nf)
        pltpu.make_async_copy(
            w1_hbm.at[en, :, pl.ds(fn_ * FB, FB)], w1buf.at[buf_slot],
            sem1.at[buf_slot]).start()
        pltpu.make_async_copy(
            w2_hbm.at[en, pl.ds(fn_ * FB, FB), :], w2buf.at[buf_slot],
            sem2.at[buf_slot]).start()

    @pl.when(s == 0)
    def _prologue():
        fetch(0, slot)

    @pl.when(s + 1 < n_e * nf)
    def _prefetch_next():
        fetch(s + 1, nslot)

    pltpu.make_async_copy(
        w1_hbm.at[0, :, pl.ds(0, FB)], w1buf.at[slot], sem1.at[slot]).wait()
    pltpu.make_async_copy(
        w2_hbm.at[0, pl.ds(0, FB), :], w2buf.at[slot], sem2.at[slot]).wait()

    w1b = w1buf[slot].astype(jnp.bfloat16)
    w2b = w2buf[slot].astype(jnp.bfloat16)

    def chunk(i, carry):
        start = pl.multiple_of(off + i * CH, 8)
        xs = x_ref[pl.ds(start, CH), :].astype(jnp.bfloat16)
        h = jnp.dot(xs, w1b, preferred_element_type=jnp.float32)
        h = jnp.maximum(h + b1_ref[0, 0], 0.0).astype(jnp.bfloat16)
        part = jnp.dot(h, w2b, preferred_element_type=jnp.float32)

        @pl.when(f == 0)
        def _init():
            out_ref[pl.ds(start, CH), :] = part + b2_ref[0]

        @pl.when((f > 0) & (f < nf - 1))
        def _acc():
            out_ref[pl.ds(start, CH), :] += part

        @pl.when(f == nf - 1)
        def _fin():
            val = out_ref[pl.ds(start, CH), :] + part
            out_ref[pl.ds(start, CH), :] = val * gate_ref[pl.ds(start, CH), :]

        return carry

    jax.lax.fori_loop(0, nc, chunk, 0)


def kernel(input_emb, Wr, br, W1, b1, W2, b2):
    B, S, D = input_emb.shape
    E = Wr.shape[1]
    F = W1.shape[2]
    nf = F // FB

    x = input_emb.reshape(S, D)
    br2 = br.reshape(1, E)
    b1r = b1.reshape(E, nf, 1, FB)
    b2r = b2.reshape(E, 1, D)

    g1, g2, i1, i2 = pl.pallas_call(
        functools.partial(_router_body, n_e=E),
        in_specs=[
            pl.BlockSpec((S, D), lambda: (0, 0)),
            pl.BlockSpec((D, E), lambda: (0, 0)),
            pl.BlockSpec((1, E), lambda: (0, 0)),
        ],
        out_specs=[
            pl.BlockSpec((S, 1), lambda: (0, 0)),
            pl.BlockSpec((S, 1), lambda: (0, 0)),
            pl.BlockSpec((S, 1), lambda: (0, 0)),
            pl.BlockSpec((S, 1), lambda: (0, 0)),
        ],
        out_shape=[
            jax.ShapeDtypeStruct((S, 1), jnp.float32),
            jax.ShapeDtypeStruct((S, 1), jnp.float32),
            jax.ShapeDtypeStruct((S, 1), jnp.int32),
            jax.ShapeDtypeStruct((S, 1), jnp.int32),
        ],
    )(x, Wr, br2)

    ee = jnp.concatenate([i1[:, 0], i2[:, 0]])
    tt = jnp.concatenate([jnp.arange(S, dtype=jnp.int32)] * 2)
    gg = jnp.concatenate([g1[:, 0], g2[:, 0]])
    onehot = (ee[:, None] == jnp.arange(E, dtype=jnp.int32)[None, :]).astype(jnp.int32)
    c = jnp.sum(onehot, axis=0)
    c8 = (c + 7) // 8 * 8
    off = jnp.concatenate([jnp.zeros(1, jnp.int32),
                           jnp.cumsum(c8)[:-1].astype(jnp.int32)])
    ranks = jnp.cumsum(onehot, axis=0) - onehot
    rank_j = jnp.take_along_axis(ranks, ee[:, None], axis=1)[:, 0]
    pos = off[ee] + rank_j
    rows = jnp.zeros((RPAD,), jnp.int32).at[pos].set(tt)
    gates = jnp.zeros((RPAD,), jnp.float32).at[pos].set(gg)
    nck = (c8 + CH - 1) // CH

    x_sorted = jnp.take(x, rows, axis=0)

    ys = pl.pallas_call(
        functools.partial(_ffn_body, nf=nf, n_e=E),
        grid=(E, nf),
        in_specs=[
            pl.BlockSpec(memory_space=pltpu.SMEM),
            pl.BlockSpec(memory_space=pltpu.SMEM),
            pl.BlockSpec((RPAD, D), lambda e, f: (0, 0)),
            pl.BlockSpec(memory_space=pl.ANY),
            pl.BlockSpec((1, 1, 1, FB), lambda e, f: (e, f, 0, 0)),
            pl.BlockSpec(memory_space=pl.ANY),
            pl.BlockSpec((1, 1, D), lambda e, f: (e, 0, 0)),
            pl.BlockSpec((RPAD, 1), lambda e, f: (0, 0)),
        ],
        out_specs=pl.BlockSpec((RPAD, D), lambda e, f: (0, 0)),
        out_shape=jax.ShapeDtypeStruct((RPAD, D), jnp.float32),
        scratch_shapes=[
            pltpu.VMEM((2, D, FB), jnp.float32),
            pltpu.VMEM((2, FB, D), jnp.float32),
            pltpu.SemaphoreType.DMA((2,)),
            pltpu.SemaphoreType.DMA((2,)),
        ],
        compiler_params=pltpu.CompilerParams(
            dimension_semantics=("arbitrary", "arbitrary"),
        ),
    )(off, nck, x_sorted, W1, b1r, W2, b2r, gates.reshape(RPAD, 1))

    p1 = pos[:S]
    p2 = pos[S:]
    out = jnp.take(ys, p1, axis=0) + jnp.take(ys, p2, axis=0)
    return out.reshape(B, S, D)


# speedup vs baseline: 1.0857x; 1.0857x over previous
import functools

import jax
import jax.numpy as jnp
from jax.experimental import pallas as pl
from jax.experimental.pallas import tpu as pltpu

CH = 512
FB = 512
RPAD = 4864


def _router_body(x_ref, wr_ref, br_ref, g1_ref, g2_ref, i1_ref, i2_ref, *, n_e):
    x = x_ref[...]
    logits = jnp.dot(x, wr_ref[...], preferred_element_type=jnp.float32)
    logits = logits + br_ref[...]
    m = jnp.max(logits, axis=1, keepdims=True)
    ex = jnp.exp(logits - m)
    rw = ex / jnp.sum(ex, axis=1, keepdims=True)
    iota = jax.lax.broadcasted_iota(jnp.int32, rw.shape, 1)
    m1 = jnp.max(rw, axis=1, keepdims=True)
    i1 = jnp.min(jnp.where(rw == m1, iota, n_e), axis=1, keepdims=True)
    oh1 = iota == i1
    rw2 = jnp.where(oh1, -1e30, rw)
    m2 = jnp.max(rw2, axis=1, keepdims=True)
    i2 = jnp.min(jnp.where(rw2 == m2, iota, n_e), axis=1, keepdims=True)
    v1 = jnp.sum(jnp.where(oh1, rw, 0.0), axis=1, keepdims=True)
    v2 = jnp.sum(jnp.where(iota == i2, rw, 0.0), axis=1, keepdims=True)
    tot = v1 + v2
    g1_ref[...] = v1 / tot
    g2_ref[...] = v2 / tot
    i1_ref[...] = i1
    i2_ref[...] = i2


def _ffn_body(off_ref, nck_ref, x_ref, w1_hbm, b1_ref, w2_hbm, b2_ref,
              gate_ref, out_ref, w1buf, w2buf, sem1, sem2, *, nf, n_e):
    e = pl.program_id(0)
    f = pl.program_id(1)
    off = off_ref[e]
    nc = nck_ref[e]
    s = e * nf + f
    slot = jax.lax.rem(s, 3)
    nslot = jax.lax.rem(s + 2, 3)

    def fetch(step, buf_slot):
        en = step // nf
        fn_ = jax.lax.rem(step, nf)
        pltpu.make_async_copy(
            w1_hbm.at[en, :, pl.ds(fn_ * FB, FB)], w1buf.at[buf_slot],
            sem1.at[buf_slot]).start()
        pltpu.make_async_copy(
            w2_hbm.at[en, pl.ds(fn_ * FB, FB), :], w2buf.at[buf_slot],
            sem2.at[buf_slot]).start()

    @pl.when(s == 0)
    def _prologue():
        fetch(0, 0)
        fetch(1, 1)

    @pl.when(s + 2 < n_e * nf)
    def _prefetch_next():
        fetch(s + 2, nslot)

    pltpu.make_async_copy(
        w1_hbm.at[0, :, pl.ds(0, FB)], w1buf.at[slot], sem1.at[slot]).wait()
    pltpu.make_async_copy(
        w2_hbm.at[0, pl.ds(0, FB), :], w2buf.at[slot], sem2.at[slot]).wait()


    def chunk(i, carry):
        start = pl.multiple_of(off + i * CH, 8)
        xs = x_ref[pl.ds(start, CH), :]
        h = jnp.dot(xs, w1buf[slot], preferred_element_type=jnp.float32)
        h = jnp.maximum(h + b1_ref[0, 0], 0.0)
        part = jnp.dot(h, w2buf[slot], preferred_element_type=jnp.float32)

        @pl.when(f == 0)
        def _init():
            out_ref[pl.ds(start, CH), :] = part + b2_ref[0]

        @pl.when((f > 0) & (f < nf - 1))
        def _acc():
            out_ref[pl.ds(start, CH), :] += part

        @pl.when(f == nf - 1)
        def _fin():
            val = out_ref[pl.ds(start, CH), :] + part
            out_ref[pl.ds(start, CH), :] = val * gate_ref[pl.ds(start, CH), :]

        return carry

    jax.lax.fori_loop(0, nc, chunk, 0)


def kernel(input_emb, Wr, br, W1, b1, W2, b2):
    B, S, D = input_emb.shape
    E = Wr.shape[1]
    F = W1.shape[2]
    nf = F // FB

    x = input_emb.reshape(S, D)
    br2 = br.reshape(1, E)
    b1r = b1.reshape(E, nf, 1, FB)
    b2r = b2.reshape(E, 1, D)

    g1, g2, i1, i2 = pl.pallas_call(
        functools.partial(_router_body, n_e=E),
        in_specs=[
            pl.BlockSpec((S, D), lambda: (0, 0)),
            pl.BlockSpec((D, E), lambda: (0, 0)),
            pl.BlockSpec((1, E), lambda: (0, 0)),
        ],
        out_specs=[
            pl.BlockSpec((S, 1), lambda: (0, 0)),
            pl.BlockSpec((S, 1), lambda: (0, 0)),
            pl.BlockSpec((S, 1), lambda: (0, 0)),
            pl.BlockSpec((S, 1), lambda: (0, 0)),
        ],
        out_shape=[
            jax.ShapeDtypeStruct((S, 1), jnp.float32),
            jax.ShapeDtypeStruct((S, 1), jnp.float32),
            jax.ShapeDtypeStruct((S, 1), jnp.int32),
            jax.ShapeDtypeStruct((S, 1), jnp.int32),
        ],
    )(x, Wr, br2)

    ee = jnp.concatenate([i1[:, 0], i2[:, 0]])
    tt = jnp.concatenate([jnp.arange(S, dtype=jnp.int32)] * 2)
    gg = jnp.concatenate([g1[:, 0], g2[:, 0]])
    onehot = (ee[:, None] == jnp.arange(E, dtype=jnp.int32)[None, :]).astype(jnp.int32)
    c = jnp.sum(onehot, axis=0)
    c8 = (c + 7) // 8 * 8
    off = jnp.concatenate([jnp.zeros(1, jnp.int32),
                           jnp.cumsum(c8)[:-1].astype(jnp.int32)])
    ranks = jnp.cumsum(onehot, axis=0) - onehot
    rank_j = jnp.take_along_axis(ranks, ee[:, None], axis=1)[:, 0]
    pos = off[ee] + rank_j
    rows = jnp.zeros((RPAD,), jnp.int32).at[pos].set(tt)
    gates = jnp.zeros((RPAD,), jnp.float32).at[pos].set(gg)
    nck = (c8 + CH - 1) // CH

    x_sorted = jnp.take(x, rows, axis=0)

    ys = pl.pallas_call(
        functools.partial(_ffn_body, nf=nf, n_e=E),
        grid=(E, nf),
        in_specs=[
            pl.BlockSpec(memory_space=pltpu.SMEM),
            pl.BlockSpec(memory_space=pltpu.SMEM),
            pl.BlockSpec((RPAD, D), lambda e, f: (0, 0)),
            pl.BlockSpec(memory_space=pl.ANY),
            pl.BlockSpec((1, 1, 1, FB), lambda e, f: (e, f, 0, 0)),
            pl.BlockSpec(memory_space=pl.ANY),
            pl.BlockSpec((1, 1, D), lambda e, f: (e, 0, 0)),
            pl.BlockSpec((RPAD, 1), lambda e, f: (0, 0)),
        ],
        out_specs=pl.BlockSpec((RPAD, D), lambda e, f: (0, 0)),
        out_shape=jax.ShapeDtypeStruct((RPAD, D), jnp.float32),
        scratch_shapes=[
            pltpu.VMEM((3, D, FB), jnp.float32),
            pltpu.VMEM((3, FB, D), jnp.float32),
            pltpu.SemaphoreType.DMA((3,)),
            pltpu.SemaphoreType.DMA((3,)),
        ],
        compiler_params=pltpu.CompilerParams(
            dimension_semantics=("arbitrary", "arbitrary"),
        ),
    )(off, nck, x_sorted, W1, b1r, W2, b2r, gates.reshape(RPAD, 1))

    p1 = pos[:S]
    p2 = pos[S:]
    out = jnp.take(ys, p1, axis=0) + jnp.take(ys, p2, axis=0)
    return out.reshape(B, S, D)


# speedup vs baseline: 1.1394x; 1.0495x over previous
import functools

import jax
import jax.numpy as jnp
from jax.experimental import pallas as pl
from jax.experimental.pallas import tpu as pltpu

CH = 512
FB = 512
RPAD = 4864


def _router_body(x_ref, wr_ref, br_ref, g1_ref, g2_ref, p1_ref, p2_ref,
                 off_ref, nck_ref, *, n_e):
    x = x_ref[...]
    s = x.shape[0]
    logits = jnp.dot(x, wr_ref[...], preferred_element_type=jnp.float32)
    logits = logits + br_ref[...]
    m = jnp.max(logits, axis=1, keepdims=True)
    ex = jnp.exp(logits - m)
    rw = ex / jnp.sum(ex, axis=1, keepdims=True)
    iota = jax.lax.broadcasted_iota(jnp.int32, rw.shape, 1)
    m1 = jnp.max(rw, axis=1, keepdims=True)
    i1 = jnp.min(jnp.where(rw == m1, iota, n_e), axis=1, keepdims=True)
    oh1 = iota == i1
    rw2 = jnp.where(oh1, -1e30, rw)
    m2 = jnp.max(rw2, axis=1, keepdims=True)
    i2 = jnp.min(jnp.where(rw2 == m2, iota, n_e), axis=1, keepdims=True)
    oh2 = iota == i2
    v1 = jnp.sum(jnp.where(oh1, rw, 0.0), axis=1, keepdims=True)
    v2 = jnp.sum(jnp.where(oh2, rw, 0.0), axis=1, keepdims=True)
    tot = v1 + v2
    g1_ref[...] = v1 / tot
    g2_ref[...] = v2 / tot

    c1 = oh1.astype(jnp.int32)
    c2 = oh2.astype(jnp.int32)
    oh1i = c1
    oh2i = c2
    k = 1
    while k < s:
        z = jnp.zeros((k, n_e), jnp.int32)
        c1 = c1 + jnp.concatenate([z, c1[:-k]], axis=0)
        c2 = c2 + jnp.concatenate([z, c2[:-k]], axis=0)
        k *= 2
    rank1 = c1 - oh1i
    rank2 = c2 - oh2i
    tot1 = c1[s - 1:s, :]
    tot2 = c2[s - 1:s, :]
    c = tot1 + tot2
    c8 = (c + 7) // 8 * 8
    offv = c8
    k = 1
    while k < n_e:
        offv = offv + jnp.concatenate(
            [jnp.zeros((1, k), jnp.int32), offv[:, :-k]], axis=1)
        k *= 2
    off_x = offv - c8

    def lane_sel(oh, arr):
        return jnp.sum(jnp.where(oh, arr, 0), axis=1, keepdims=True)

    p1_ref[...] = lane_sel(oh1, off_x + rank1)
    p2_ref[...] = lane_sel(oh2, off_x + tot1 + rank2)
    off_ref[...] = off_x
    nck_ref[...] = (c8 + CH - 1) // CH


def _ffn_body(off_ref, nck_ref, x_ref, w1_hbm, b1_ref, w2_hbm, b2_ref,
              gate_ref, out_ref, w1buf, w2buf, sem1, sem2, *, nf, n_e):
    e = pl.program_id(0)
    f = pl.program_id(1)
    off = off_ref[e]
    nc = nck_ref[e]
    s = e * nf + f
    slot = jax.lax.rem(s, 3)
    nslot = jax.lax.rem(s + 2, 3)

    def fetch(step, buf_slot):
        en = step // nf
        fn_ = jax.lax.rem(step, nf)
        pltpu.make_async_copy(
            w1_hbm.at[en, :, pl.ds(fn_ * FB, FB)], w1buf.at[buf_slot],
            sem1.at[buf_slot]).start()
        pltpu.make_async_copy(
            w2_hbm.at[en, pl.ds(fn_ * FB, FB), :], w2buf.at[buf_slot],
            sem2.at[buf_slot]).start()

    @pl.when(s == 0)
    def _prologue():
        fetch(0, 0)
        fetch(1, 1)

    @pl.when(s + 2 < n_e * nf)
    def _prefetch_next():
        fetch(s + 2, nslot)

    pltpu.make_async_copy(
        w1_hbm.at[0, :, pl.ds(0, FB)], w1buf.at[slot], sem1.at[slot]).wait()
    pltpu.make_async_copy(
        w2_hbm.at[0, pl.ds(0, FB), :], w2buf.at[slot], sem2.at[slot]).wait()


    def chunk(i, carry):
        start = pl.multiple_of(off + i * CH, 8)
        xs = x_ref[pl.ds(start, CH), :]
        h = jnp.dot(xs, w1buf[slot], preferred_element_type=jnp.float32)
        h = jnp.maximum(h + b1_ref[0, 0], 0.0)
        part = jnp.dot(h, w2buf[slot], preferred_element_type=jnp.float32)

        @pl.when(f == 0)
        def _init():
            out_ref[pl.ds(start, CH), :] = part + b2_ref[0]

        @pl.when((f > 0) & (f < nf - 1))
        def _acc():
            out_ref[pl.ds(start, CH), :] += part

        @pl.when(f == nf - 1)
        def _fin():
            val = out_ref[pl.ds(start, CH), :] + part
            out_ref[pl.ds(start, CH), :] = val * gate_ref[pl.ds(start, CH), :]

        return carry

    jax.lax.fori_loop(0, nc, chunk, 0)


def kernel(input_emb, Wr, br, W1, b1, W2, b2):
    B, S, D = input_emb.shape
    E = Wr.shape[1]
    F = W1.shape[2]
    nf = F // FB

    x = input_emb.reshape(S, D)
    br2 = br.reshape(1, E)
    b1r = b1.reshape(E, nf, 1, FB)
    b2r = b2.reshape(E, 1, D)

    g1, g2, p1c, p2c, offs, ncks = pl.pallas_call(
        functools.partial(_router_body, n_e=E),
        in_specs=[
            pl.BlockSpec((S, D), lambda: (0, 0)),
            pl.BlockSpec((D, E), lambda: (0, 0)),
            pl.BlockSpec((1, E), lambda: (0, 0)),
        ],
        out_specs=[
            pl.BlockSpec((S, 1), lambda: (0, 0)),
            pl.BlockSpec((S, 1), lambda: (0, 0)),
            pl.BlockSpec((S, 1), lambda: (0, 0)),
            pl.BlockSpec((S, 1), lambda: (0, 0)),
            pl.BlockSpec((1, E), lambda: (0, 0)),
            pl.BlockSpec((1, E), lambda: (0, 0)),
        ],
        out_shape=[
            jax.ShapeDtypeStruct((S, 1), jnp.float32),
            jax.ShapeDtypeStruct((S, 1), jnp.float32),
            jax.ShapeDtypeStruct((S, 1), jnp.int32),
            jax.ShapeDtypeStruct((S, 1), jnp.int32),
            jax.ShapeDtypeStruct((1, E), jnp.int32),
            jax.ShapeDtypeStruct((1, E), jnp.int32),
        ],
    )(x, Wr, br2)

    pos = jnp.concatenate([p1c[:, 0], p2c[:, 0]])
    tt = jnp.concatenate([jnp.arange(S, dtype=jnp.int32)] * 2)
    gg = jnp.concatenate([g1[:, 0], g2[:, 0]])
    rows = jnp.zeros((RPAD,), jnp.int32).at[pos].set(tt)
    gates = jnp.zeros((RPAD,), jnp.float32).at[pos].set(gg)
    off = offs.reshape(E)
    nck = ncks.reshape(E)
    x_sorted = jnp.take(x, rows, axis=0)

    ys = pl.pallas_call(
        functools.partial(_ffn_body, nf=nf, n_e=E),
        grid=(E, nf),
        in_specs=[
            pl.BlockSpec(memory_space=pltpu.SMEM),
            pl.BlockSpec(memory_space=pltpu.SMEM),
            pl.BlockSpec((RPAD, D), lambda e, f: (0, 0)),
            pl.BlockSpec(memory_space=pl.ANY),
            pl.BlockSpec((1, 1, 1, FB), lambda e, f: (e, f, 0, 0)),
            pl.BlockSpec(memory_space=pl.ANY),
            pl.BlockSpec((1, 1, D), lambda e, f: (e, 0, 0)),
            pl.BlockSpec((RPAD, 1), lambda e, f: (0, 0)),
        ],
        out_specs=pl.BlockSpec((RPAD, D), lambda e, f: (0, 0)),
        out_shape=jax.ShapeDtypeStruct((RPAD, D), jnp.float32),
        scratch_shapes=[
            pltpu.VMEM((3, D, FB), jnp.float32),
            pltpu.VMEM((3, FB, D), jnp.float32),
            pltpu.SemaphoreType.DMA((3,)),
            pltpu.SemaphoreType.DMA((3,)),
        ],
        compiler_params=pltpu.CompilerParams(
            dimension_semantics=("arbitrary", "arbitrary"),
        ),
    )(off, nck, x_sorted, W1, b1r, W2, b2r, gates.reshape(RPAD, 1))

    p1 = p1c[:, 0]
    p2 = p2c[:, 0]
    out = jnp.take(ys, p1, axis=0) + jnp.take(ys, p2, axis=0)
    return out.reshape(B, S, D)
